# baseline (speedup 1.0000x reference)
import functools

import jax
import jax.numpy as jnp
from jax import lax
from jax.experimental import pallas as pl
from jax.experimental.pallas import tpu as pltpu
from jax.experimental.pallas import tpu_sc as plsc

N = 10000
E = 320000
D = 128

NC = 2
NS = 16
L = 16
NW = NC * NS
EW = E // NW
CH = 40
NCH = EW // CH
RPT = N // NS

_mesh = plsc.VectorSubcoreMesh(core_axis_name="c", subcore_axis_name="s")
_sc_params = pltpu.CompilerParams(
    needs_layout_passes=False, use_tc_tiling_on_sc=False)


@functools.partial(
    pl.kernel,
    out_type=jax.ShapeDtypeStruct((2, NW, N), jnp.float32),
    mesh=_mesh,
    scratch_types=[
        pltpu.VMEM((EW,), jnp.int32),
        pltpu.VMEM((EW,), jnp.int32),
        pltpu.VMEM((N,), jnp.float32),
        pltpu.VMEM((N,), jnp.float32),
    ],
    compiler_params=_sc_params,
)
def _degrees_sc(eidx_hbm, out_hbm, sidx, didx, dego, degi):
    c = lax.axis_index("c")
    s = lax.axis_index("s")
    wid = c * NS + s
    pltpu.sync_copy(eidx_hbm.at[0, wid], sidx)
    pltpu.sync_copy(eidx_hbm.at[1, wid], didx)

    zero16 = jnp.zeros((L,), jnp.float32)

    def zb(g, carry):
        dego[pl.ds(g * L, L)] = zero16
        degi[pl.ds(g * L, L)] = zero16
        return carry

    lax.fori_loop(0, N // L, zb, 0)

    one16 = jnp.ones((L,), jnp.float32)

    def cb(g, carry):
        plsc.addupdate_scatter(dego, [sidx[pl.ds(g * L, L)]], one16)
        plsc.addupdate_scatter(degi, [didx[pl.ds(g * L, L)]], one16)
        return carry

    lax.fori_loop(0, EW // L, cb, 0)

    pltpu.sync_copy(dego, out_hbm.at[0, wid])
    pltpu.sync_copy(degi, out_hbm.at[1, wid])


def _norms_body(degp_ref, out_ref):
    deg = jnp.sum(degp_ref[...], axis=1)
    out_ref[...] = lax.rsqrt(jnp.clip(deg, 1.0, None))


def _norms_tc(degp):
    return pl.pallas_call(
        _norms_body,
        out_shape=jax.ShapeDtypeStruct((2, N), jnp.float32),
    )(degp)


RING = 5


@functools.partial(
    pl.kernel,
    out_type=jax.ShapeDtypeStruct((NC, N, D), jnp.float32),
    mesh=_mesh,
    scratch_types=(
        [
            pltpu.VMEM_SHARED((N, D), jnp.float32),
            pltpu.VMEM((N,), jnp.float32),
        ]
        + [pltpu.VMEM((CH, D), jnp.float32)] * RING
        + [pltpu.VMEM((3, CH), jnp.int32)] * RING
        + [pltpu.SemaphoreType.DMA] * (3 * RING)
    ),
    compiler_params=_sc_params,
)
def _scatter_sc(x_hbm, nrm_hbm, pk_hbm, out_hbm, agg, nrm_loc, *ring):
    rows = ring[:RING]
    er = ring[RING:2 * RING]
    semg = ring[2 * RING:3 * RING]
    sems = ring[3 * RING:4 * RING]
    sem_e = ring[4 * RING:5 * RING]
    c = lax.axis_index("c")
    s = lax.axis_index("s")
    wid = c * NS + s

    pltpu.sync_copy(nrm_hbm.at[0], nrm_loc)

    zero16 = jnp.zeros((L,), jnp.float32)

    def zrows(i, carry):
        j = i // (D // L)
        q = i % (D // L)
        rows[0][j, pl.ds(q * L, L)] = zero16
        return carry

    lax.fori_loop(0, CH * (D // L), zrows, 0)

    base = s * RPT
    for k in range(RPT // CH):
        pltpu.sync_copy(rows[0], agg.at[pl.ds(base + k * CH, CH)])
    pltpu.sync_copy(rows[0].at[pl.ds(0, RPT % CH)],
                    agg.at[pl.ds(base + (RPT // CH) * CH, RPT % CH)])

    for j in range(3):
        pltpu.async_copy(pk_hbm.at[wid, j], er[j], sem_e[j])
    for j in range(2):
        pltpu.make_async_copy(pk_hbm.at[wid, j], er[j], sem_e[j]).wait()
        pltpu.async_copy(x_hbm.at[er[j].at[0]], rows[j], semg[j])

    plsc.subcore_barrier()

    lane0 = jnp.zeros((L,), jnp.int32)
    lane2 = lane0 + 2

    def outer(i, carry):
        for j in range(RING):
            k = i * RING + j
            pltpu.make_async_copy(x_hbm.at[er[j].at[0]], rows[j], semg[j]).wait()

            def scale_row(r2, inner, j=j):
                for u in range(2):
                    r16 = lane0 + (r2 * 2 + u)
                    s16 = plsc.load_gather(er[j], [lane0, r16])
                    wi = plsc.load_gather(er[j], [lane2, r16])
                    sv = plsc.bitcast(wi, jnp.float32) \
                        * plsc.load_gather(nrm_loc, [s16])
                    for q in range(D // L):
                        rows[j][r2 * 2 + u, pl.ds(q * L, L)] = (
                            rows[j][r2 * 2 + u, pl.ds(q * L, L)] * sv)
                return inner

            lax.fori_loop(0, CH // 2, scale_row, 0)
            pltpu.async_copy(rows[j], agg.at[er[j].at[1]], sems[j], add=True)

            j2 = (j + RING - 2) % RING
            j1 = (j + 2) % RING

            @pl.when(k >= 2)
            def _():
                pltpu.make_async_copy(
                    rows[j2], agg.at[er[j2].at[1]], sems[j2]).wait()

            @pl.when(k + 3 < NCH)
            def _():
                pltpu.async_copy(pk_hbm.at[wid, k + 3], er[j2], sem_e[j2])

            @pl.when(k + 2 < NCH)
            def _():
                pltpu.make_async_copy(
                    pk_hbm.at[wid, k + 2], er[j1], sem_e[j1]).wait()
                pltpu.async_copy(x_hbm.at[er[j1].at[0]], rows[j1], semg[j1])
        return carry

    lax.fori_loop(0, NCH // RING, outer, 0)
    for j in (RING - 2, RING - 1):
        pltpu.make_async_copy(rows[j], agg.at[er[j].at[1]], sems[j]).wait()
    plsc.subcore_barrier()
    pltpu.sync_copy(agg.at[pl.ds(base, RPT)], out_hbm.at[c, pl.ds(base, RPT)])


BLK = 400


def _final_body(agg_ref, w_ref, nd_ref, b_ref, out_ref):
    a = agg_ref[0] + agg_ref[1]
    acc = jnp.dot(a, w_ref[...], preferred_element_type=jnp.float32)
    out_ref[...] = acc * nd_ref[...] + b_ref[...]


def _final_tc(aggp, W, ndst, b):
    return pl.pallas_call(
        _final_body,
        grid=(N // BLK,),
        in_specs=[
            pl.BlockSpec((2, BLK, D), lambda i: (0, i, 0)),
            pl.BlockSpec((D, D), lambda i: (0, 0)),
            pl.BlockSpec((BLK, 1), lambda i: (i, 0)),
            pl.BlockSpec((1, D), lambda i: (0, 0)),
        ],
        out_specs=pl.BlockSpec((BLK, D), lambda i: (i, 0)),
        out_shape=jax.ShapeDtypeStruct((N, D), jnp.float32),
    )(aggp, W, ndst, b.reshape(1, D))


def kernel(node_embedding, edge_embedding, edge_index, W, b):
    eidx = edge_index.astype(jnp.int32)
    e2 = eidx.reshape(2, NW, NCH, 1, CH)
    wbits = lax.bitcast_convert_type(
        edge_embedding.astype(jnp.float32).reshape(NW, NCH, 1, CH), jnp.int32)
    pk = jnp.concatenate([e2[0], e2[1], wbits], axis=2)

    degp = _degrees_sc(eidx.reshape(2, NW, EW))
    norms = _norms_tc(degp)
    aggp = _scatter_sc(node_embedding, norms, pk)
    return _final_tc(aggp, W, norms[1].reshape(N, 1), b)


# speedup vs baseline: 1.3454x; 1.3454x over previous
import functools

import jax
import jax.numpy as jnp
from jax import lax
from jax.experimental import pallas as pl
from jax.experimental.pallas import tpu as pltpu
from jax.experimental.pallas import tpu_sc as plsc

N = 10000
E = 320000
D = 128

NC = 2
NS = 16
L = 16
NW = NC * NS
EW = E // NW
CH = 40
NCH = EW // CH
RPT = N // NS

_mesh = plsc.VectorSubcoreMesh(core_axis_name="c", subcore_axis_name="s")
_sc_params = pltpu.CompilerParams(
    needs_layout_passes=False, use_tc_tiling_on_sc=False)


@functools.partial(
    pl.kernel,
    out_type=jax.ShapeDtypeStruct((2, NW, N), jnp.float32),
    mesh=_mesh,
    scratch_types=[
        pltpu.VMEM((EW,), jnp.int32),
        pltpu.VMEM((EW,), jnp.int32),
        pltpu.VMEM((N,), jnp.float32),
        pltpu.VMEM((N,), jnp.float32),
    ],
    compiler_params=_sc_params,
)
def _degrees_sc(eidx_hbm, out_hbm, sidx, didx, dego, degi):
    c = lax.axis_index("c")
    s = lax.axis_index("s")
    wid = c * NS + s
    pltpu.sync_copy(eidx_hbm.at[0, wid], sidx)
    pltpu.sync_copy(eidx_hbm.at[1, wid], didx)

    zero16 = jnp.zeros((L,), jnp.float32)

    def zb(g, carry):
        dego[pl.ds(g * L, L)] = zero16
        degi[pl.ds(g * L, L)] = zero16
        return carry

    lax.fori_loop(0, N // L, zb, 0)

    one16 = jnp.ones((L,), jnp.float32)

    def cb(g, carry):
        plsc.addupdate_scatter(dego, [sidx[pl.ds(g * L, L)]], one16)
        plsc.addupdate_scatter(degi, [didx[pl.ds(g * L, L)]], one16)
        return carry

    lax.fori_loop(0, EW // L, cb, 0)

    pltpu.sync_copy(dego, out_hbm.at[0, wid])
    pltpu.sync_copy(degi, out_hbm.at[1, wid])


def _norms_body(degp_ref, out_ref):
    deg = jnp.sum(degp_ref[...], axis=1)
    out_ref[...] = lax.rsqrt(jnp.clip(deg, 1.0, None))


def _norms_tc(degp):
    return pl.pallas_call(
        _norms_body,
        out_shape=jax.ShapeDtypeStruct((2, N), jnp.float32),
    )(degp)


RING = 5


@functools.partial(
    pl.kernel,
    out_type=jax.ShapeDtypeStruct((NC, N, D), jnp.float32),
    mesh=_mesh,
    scratch_types=(
        [
            pltpu.VMEM_SHARED((N, D), jnp.float32),
            pltpu.VMEM((N,), jnp.float32),
        ]
        + [pltpu.VMEM((CH, D), jnp.float32)] * RING
        + [pltpu.VMEM((1, CH), jnp.int32)] * RING
        + [pltpu.VMEM((1, CH), jnp.int32)] * RING
        + [pltpu.VMEM((CH,), jnp.float32)] * RING
        + [pltpu.SemaphoreType.DMA] * (5 * RING)
    ),
    compiler_params=_sc_params,
)
def _scatter_sc(x_hbm, nrm_hbm, eidx_hbm, w_hbm, out_hbm, agg,
                nrm_loc, *ring):
    rows = ring[:RING]
    sr = ring[RING:2 * RING]
    dr = ring[2 * RING:3 * RING]
    wr = ring[3 * RING:4 * RING]
    semg = ring[4 * RING:5 * RING]
    sems = ring[5 * RING:6 * RING]
    sem_s = ring[6 * RING:7 * RING]
    sem_d = ring[7 * RING:8 * RING]
    sem_w = ring[8 * RING:9 * RING]
    c = lax.axis_index("c")
    s = lax.axis_index("s")
    wid = c * NS + s

    pltpu.sync_copy(nrm_hbm.at[0], nrm_loc)

    zero16 = jnp.zeros((L,), jnp.float32)

    def zrows(i, carry):
        j = i // (D // L)
        q = i % (D // L)
        rows[0][j, pl.ds(q * L, L)] = zero16
        return carry

    lax.fori_loop(0, CH * (D // L), zrows, 0)

    base = s * RPT
    for k in range(RPT // CH):
        pltpu.sync_copy(rows[0], agg.at[pl.ds(base + k * CH, CH)])
    pltpu.sync_copy(rows[0].at[pl.ds(0, RPT % CH)],
                    agg.at[pl.ds(base + (RPT // CH) * CH, RPT % CH)])

    for j in range(3):
        pltpu.async_copy(eidx_hbm.at[0, wid, j], sr[j], sem_s[j])
        pltpu.async_copy(eidx_hbm.at[1, wid, j], dr[j], sem_d[j])
        pltpu.async_copy(w_hbm.at[wid, j], wr[j], sem_w[j])
    for j in range(2):
        pltpu.make_async_copy(eidx_hbm.at[0, wid, j], sr[j], sem_s[j]).wait()
        pltpu.async_copy(x_hbm.at[sr[j].at[0]], rows[j], semg[j])

    plsc.subcore_barrier()

    lane0 = jnp.zeros((L,), jnp.int32)

    def outer(i, carry):
        for j in range(RING):
            k = i * RING + j
            pltpu.make_async_copy(x_hbm.at[sr[j].at[0]], rows[j], semg[j]).wait()
            pltpu.make_async_copy(eidx_hbm.at[1, wid, k], dr[j], sem_d[j]).wait()
            pltpu.make_async_copy(w_hbm.at[wid, k], wr[j], sem_w[j]).wait()

            def scale_row(r2, inner, j=j):
                for u in range(2):
                    r = r2 * 2 + u
                    r16 = lane0 + r
                    s16 = plsc.load_gather(sr[j], [lane0, r16])
                    w16 = plsc.load_gather(wr[j], [r16])
                    sv = w16 * plsc.load_gather(nrm_loc, [s16])
                    for q in range(D // L):
                        rows[j][r, pl.ds(q * L, L)] = (
                            rows[j][r, pl.ds(q * L, L)] * sv)
                return inner

            lax.fori_loop(0, CH // 2, scale_row, 0)
            pltpu.async_copy(rows[j], agg.at[dr[j].at[0]], sems[j], add=True)

            j2 = (j + RING - 2) % RING
            j1 = (j + 2) % RING

            @pl.when(k >= 2)
            def _():
                pltpu.make_async_copy(
                    rows[j2], agg.at[dr[j2].at[0]], sems[j2]).wait()

            @pl.when(k + 3 < NCH)
            def _():
                pltpu.async_copy(eidx_hbm.at[0, wid, k + 3], sr[j2], sem_s[j2])
                pltpu.async_copy(eidx_hbm.at[1, wid, k + 3], dr[j2], sem_d[j2])
                pltpu.async_copy(w_hbm.at[wid, k + 3], wr[j2], sem_w[j2])

            @pl.when(k + 2 < NCH)
            def _():
                pltpu.make_async_copy(
                    eidx_hbm.at[0, wid, k + 2], sr[j1], sem_s[j1]).wait()
                pltpu.async_copy(x_hbm.at[sr[j1].at[0]], rows[j1], semg[j1])
        return carry

    lax.fori_loop(0, NCH // RING, outer, 0)
    for j in (RING - 2, RING - 1):
        pltpu.make_async_copy(rows[j], agg.at[dr[j].at[0]], sems[j]).wait()
    plsc.subcore_barrier()
    pltpu.sync_copy(agg.at[pl.ds(base, RPT)], out_hbm.at[c, pl.ds(base, RPT)])


BLK = 400


def _final_body(agg_ref, w_ref, nd_ref, b_ref, out_ref):
    a = agg_ref[0] + agg_ref[1]
    acc = jnp.dot(a, w_ref[...], preferred_element_type=jnp.float32)
    out_ref[...] = acc * nd_ref[...] + b_ref[...]


def _final_tc(aggp, W, ndst, b):
    return pl.pallas_call(
        _final_body,
        grid=(N // BLK,),
        in_specs=[
            pl.BlockSpec((2, BLK, D), lambda i: (0, i, 0)),
            pl.BlockSpec((D, D), lambda i: (0, 0)),
            pl.BlockSpec((BLK, 1), lambda i: (i, 0)),
            pl.BlockSpec((1, D), lambda i: (0, 0)),
        ],
        out_specs=pl.BlockSpec((BLK, D), lambda i: (i, 0)),
        out_shape=jax.ShapeDtypeStruct((N, D), jnp.float32),
    )(aggp, W, ndst, b.reshape(1, D))


def kernel(node_embedding, edge_embedding, edge_index, W, b):
    eidx = edge_index.astype(jnp.int32)
    w3 = edge_embedding.astype(jnp.float32).reshape(NW, NCH, CH)

    degp = _degrees_sc(eidx.reshape(2, NW, EW))
    norms = _norms_tc(degp)
    aggp = _scatter_sc(
        node_embedding, norms, eidx.reshape(2, NW, NCH, 1, CH), w3)
    return _final_tc(aggp, W, norms[1].reshape(N, 1), b)


# speedup vs baseline: 1.4581x; 1.0838x over previous
import functools

import jax
import jax.numpy as jnp
from jax import lax
from jax.experimental import pallas as pl
from jax.experimental.pallas import tpu as pltpu
from jax.experimental.pallas import tpu_sc as plsc

N = 10000
E = 320000
D = 128

NC = 2
NS = 16
L = 16
NW = NC * NS
EW = E // NW
CH = 40
NCH = EW // CH
RPT = N // NS

_mesh = plsc.VectorSubcoreMesh(core_axis_name="c", subcore_axis_name="s")
_sc_params = pltpu.CompilerParams(
    needs_layout_passes=False, use_tc_tiling_on_sc=False)


@functools.partial(
    pl.kernel,
    out_type=jax.ShapeDtypeStruct((2, NW, N), jnp.float32),
    mesh=_mesh,
    scratch_types=[
        pltpu.VMEM((EW,), jnp.int32),
        pltpu.VMEM((EW,), jnp.int32),
        pltpu.VMEM((N,), jnp.float32),
        pltpu.VMEM((N,), jnp.float32),
    ],
    compiler_params=_sc_params,
)
def _degrees_sc(eidx_hbm, out_hbm, sidx, didx, dego, degi):
    c = lax.axis_index("c")
    s = lax.axis_index("s")
    wid = c * NS + s
    pltpu.sync_copy(eidx_hbm.at[0, wid], sidx)
    pltpu.sync_copy(eidx_hbm.at[1, wid], didx)

    zero16 = jnp.zeros((L,), jnp.float32)

    def zb(g, carry):
        dego[pl.ds(g * L, L)] = zero16
        degi[pl.ds(g * L, L)] = zero16
        return carry

    lax.fori_loop(0, N // L, zb, 0)

    one16 = jnp.ones((L,), jnp.float32)

    def cb(g, carry):
        plsc.addupdate_scatter(dego, [sidx[pl.ds(g * L, L)]], one16)
        plsc.addupdate_scatter(degi, [didx[pl.ds(g * L, L)]], one16)
        return carry

    lax.fori_loop(0, EW // L, cb, 0)

    pltpu.sync_copy(dego, out_hbm.at[0, wid])
    pltpu.sync_copy(degi, out_hbm.at[1, wid])


def _norms_body(degp_ref, out_ref):
    deg = jnp.sum(degp_ref[...], axis=1)
    out_ref[...] = lax.rsqrt(jnp.clip(deg, 1.0, None))


def _norms_tc(degp):
    return pl.pallas_call(
        _norms_body,
        out_shape=jax.ShapeDtypeStruct((2, N), jnp.float32),
    )(degp)


def _feat_body(x_ref, ns_ref, feat_ref):
    feat_ref[...] = x_ref[...] * ns_ref[...]


def _feat_tc(x, nsrc):
    return pl.pallas_call(
        _feat_body,
        out_shape=jax.ShapeDtypeStruct((N, D), jnp.float32),
    )(x, nsrc)


RING = 5


@functools.partial(
    pl.kernel,
    out_type=jax.ShapeDtypeStruct((NC, N, D), jnp.float32),
    mesh=_mesh,
    scratch_types=(
        [
            pltpu.VMEM_SHARED((N, D), jnp.float32),
            pltpu.VMEM((NCH, 1, CH), jnp.int32),
            pltpu.VMEM((EW,), jnp.float32),
        ]
        + [pltpu.VMEM((CH, D), jnp.float32)] * RING
        + [pltpu.VMEM((1, CH), jnp.int32)] * RING
        + [pltpu.SemaphoreType.DMA] * (3 * RING)
    ),
    compiler_params=_sc_params,
)
def _scatter_sc(x_hbm, eidx_hbm, w_hbm, out_hbm, agg, dst2d, w_loc, *ring):
    rows = ring[:RING]
    sr = ring[RING:2 * RING]
    semg = ring[2 * RING:3 * RING]
    sems = ring[3 * RING:4 * RING]
    sem_s = ring[4 * RING:5 * RING]
    c = lax.axis_index("c")
    s = lax.axis_index("s")
    wid = c * NS + s

    pltpu.sync_copy(eidx_hbm.at[1, wid], dst2d)
    pltpu.sync_copy(w_hbm.at[wid], w_loc)

    zero16 = jnp.zeros((L,), jnp.float32)

    def zrows(i, carry):
        j = i // (D // L)
        q = i % (D // L)
        rows[0][j, pl.ds(q * L, L)] = zero16
        return carry

    lax.fori_loop(0, CH * (D // L), zrows, 0)

    base = s * RPT
    for k in range(RPT // CH):
        pltpu.sync_copy(rows[0], agg.at[pl.ds(base + k * CH, CH)])
    pltpu.sync_copy(rows[0].at[pl.ds(0, RPT % CH)],
                    agg.at[pl.ds(base + (RPT // CH) * CH, RPT % CH)])

    for j in range(3):
        pltpu.async_copy(eidx_hbm.at[0, wid, j], sr[j], sem_s[j])
    for j in range(2):
        pltpu.make_async_copy(eidx_hbm.at[0, wid, j], sr[j], sem_s[j]).wait()
        pltpu.async_copy(x_hbm.at[sr[j].at[0]], rows[j], semg[j])

    plsc.subcore_barrier()

    lane0 = jnp.zeros((L,), jnp.int32)

    def outer(i, carry):
        for j in range(RING):
            k = i * RING + j
            pltpu.make_async_copy(x_hbm.at[sr[j].at[0]], rows[j], semg[j]).wait()

            def scale_row(r2, inner, j=j, k=k):
                for u in range(2):
                    r = r2 * 2 + u
                    sv = plsc.load_gather(w_loc, [lane0 + (k * CH + r)])
                    for q in range(D // L):
                        rows[j][r, pl.ds(q * L, L)] = (
                            rows[j][r, pl.ds(q * L, L)] * sv)
                return inner

            lax.fori_loop(0, CH // 2, scale_row, 0)
            pltpu.async_copy(rows[j], agg.at[dst2d.at[k, 0]], sems[j], add=True)

            j2 = (j + RING - 2) % RING
            j1 = (j + 2) % RING

            @pl.when(k >= 2)
            def _():
                pltpu.make_async_copy(
                    rows[j2], agg.at[dst2d.at[k - 2, 0]], sems[j2]).wait()

            @pl.when(k + 3 < NCH)
            def _():
                pltpu.async_copy(eidx_hbm.at[0, wid, k + 3], sr[j2], sem_s[j2])

            @pl.when(k + 2 < NCH)
            def _():
                pltpu.make_async_copy(
                    eidx_hbm.at[0, wid, k + 2], sr[j1], sem_s[j1]).wait()
                pltpu.async_copy(x_hbm.at[sr[j1].at[0]], rows[j1], semg[j1])
        return carry

    lax.fori_loop(0, NCH // RING, outer, 0)
    for j in (RING - 2, RING - 1):
        pltpu.make_async_copy(
            rows[j], agg.at[dst2d.at[NCH - RING + j, 0]], sems[j]).wait()
    plsc.subcore_barrier()
    pltpu.sync_copy(agg.at[pl.ds(base, RPT)], out_hbm.at[c, pl.ds(base, RPT)])


BLK = 400


def _final_body(agg_ref, w_ref, nd_ref, b_ref, out_ref):
    a = agg_ref[0] + agg_ref[1]
    acc = jnp.dot(a, w_ref[...], preferred_element_type=jnp.float32)
    out_ref[...] = acc * nd_ref[...] + b_ref[...]


def _final_tc(aggp, W, ndst, b):
    return pl.pallas_call(
        _final_body,
        grid=(N // BLK,),
        in_specs=[
            pl.BlockSpec((2, BLK, D), lambda i: (0, i, 0)),
            pl.BlockSpec((D, D), lambda i: (0, 0)),
            pl.BlockSpec((BLK, 1), lambda i: (i, 0)),
            pl.BlockSpec((1, D), lambda i: (0, 0)),
        ],
        out_specs=pl.BlockSpec((BLK, D), lambda i: (i, 0)),
        out_shape=jax.ShapeDtypeStruct((N, D), jnp.float32),
    )(aggp, W, ndst, b.reshape(1, D))


def kernel(node_embedding, edge_embedding, edge_index, W, b):
    eidx = edge_index.astype(jnp.int32)
    w3 = edge_embedding.astype(jnp.float32).reshape(NW, EW)

    degp = _degrees_sc(eidx.reshape(2, NW, EW))
    norms = _norms_tc(degp)
    feat = _feat_tc(node_embedding, norms[0].reshape(N, 1))
    aggp = _scatter_sc(feat, eidx.reshape(2, NW, NCH, 1, CH), w3)
    return _final_tc(aggp, W, norms[1].reshape(N, 1), b)


# speedup vs baseline: 1.5079x; 1.0342x over previous
import functools

import jax
import jax.numpy as jnp
from jax import lax
from jax.experimental import pallas as pl
from jax.experimental.pallas import tpu as pltpu
from jax.experimental.pallas import tpu_sc as plsc

N = 10000
E = 320000
D = 128

NC = 2
NS = 16
L = 16
NW = NC * NS
EW = E // NW
CH = 40
NCH = EW // CH
RPT = N // NS

_mesh = plsc.VectorSubcoreMesh(core_axis_name="c", subcore_axis_name="s")
_sc_params = pltpu.CompilerParams(
    needs_layout_passes=False, use_tc_tiling_on_sc=False)


@functools.partial(
    pl.kernel,
    out_type=jax.ShapeDtypeStruct((2, NW, N), jnp.float32),
    mesh=_mesh,
    scratch_types=[
        pltpu.VMEM((EW,), jnp.int32),
        pltpu.VMEM((EW,), jnp.int32),
        pltpu.VMEM((N,), jnp.float32),
        pltpu.VMEM((N,), jnp.float32),
    ],
    compiler_params=_sc_params,
)
def _degrees_sc(eidx_hbm, out_hbm, sidx, didx, dego, degi):
    c = lax.axis_index("c")
    s = lax.axis_index("s")
    wid = c * NS + s
    pltpu.sync_copy(eidx_hbm.at[0, wid], sidx)
    pltpu.sync_copy(eidx_hbm.at[1, wid], didx)

    zero16 = jnp.zeros((L,), jnp.float32)

    def zb(g, carry):
        dego[pl.ds(g * L, L)] = zero16
        degi[pl.ds(g * L, L)] = zero16
        return carry

    lax.fori_loop(0, N // L, zb, 0)

    one16 = jnp.ones((L,), jnp.float32)

    def cb(g, carry):
        for u in range(2):
            o = (g * 2 + u) * L
            plsc.addupdate_scatter(dego, [sidx[pl.ds(o, L)]], one16)
            plsc.addupdate_scatter(degi, [didx[pl.ds(o, L)]], one16)
        return carry

    lax.fori_loop(0, EW // (2 * L), cb, 0)

    pltpu.sync_copy(dego, out_hbm.at[0, wid])
    pltpu.sync_copy(degi, out_hbm.at[1, wid])


def _norms_body(degp_ref, out_ref):
    deg = jnp.sum(degp_ref[...], axis=1)
    out_ref[...] = lax.rsqrt(jnp.clip(deg, 1.0, None))


def _norms_tc(degp):
    return pl.pallas_call(
        _norms_body,
        out_shape=jax.ShapeDtypeStruct((2, N), jnp.float32),
    )(degp)


def _feat_body(x_ref, ns_ref, feat_ref):
    feat_ref[...] = x_ref[...] * ns_ref[...]


def _feat_tc(x, nsrc):
    return pl.pallas_call(
        _feat_body,
        out_shape=jax.ShapeDtypeStruct((N, D), jnp.float32),
    )(x, nsrc)


RING = 5


@functools.partial(
    pl.kernel,
    out_type=jax.ShapeDtypeStruct((NC, N, D), jnp.float32),
    mesh=_mesh,
    scratch_types=(
        [
            pltpu.VMEM_SHARED((N, D), jnp.float32),
            pltpu.VMEM((NCH, 1, CH), jnp.int32),
            pltpu.VMEM((EW,), jnp.float32),
        ]
        + [pltpu.VMEM((CH, D), jnp.float32)] * RING
        + [pltpu.VMEM((1, CH), jnp.int32)] * RING
        + [pltpu.SemaphoreType.DMA] * (3 * RING)
    ),
    compiler_params=_sc_params,
)
def _scatter_sc(x_hbm, eidx_hbm, w_hbm, out_hbm, agg, dst2d, w_loc, *ring):
    rows = ring[:RING]
    sr = ring[RING:2 * RING]
    semg = ring[2 * RING:3 * RING]
    sems = ring[3 * RING:4 * RING]
    sem_s = ring[4 * RING:5 * RING]
    c = lax.axis_index("c")
    s = lax.axis_index("s")
    wid = c * NS + s

    pltpu.sync_copy(eidx_hbm.at[1, wid], dst2d)
    pltpu.sync_copy(w_hbm.at[wid], w_loc)

    zero16 = jnp.zeros((L,), jnp.float32)

    def zrows(i, carry):
        j = i // (D // L)
        q = i % (D // L)
        rows[0][j, pl.ds(q * L, L)] = zero16
        return carry

    lax.fori_loop(0, CH * (D // L), zrows, 0)

    base = s * RPT
    for k in range(RPT // CH):
        pltpu.sync_copy(rows[0], agg.at[pl.ds(base + k * CH, CH)])
    pltpu.sync_copy(rows[0].at[pl.ds(0, RPT % CH)],
                    agg.at[pl.ds(base + (RPT // CH) * CH, RPT % CH)])

    for j in range(3):
        pltpu.async_copy(eidx_hbm.at[0, wid, j], sr[j], sem_s[j])
    for j in range(2):
        pltpu.make_async_copy(eidx_hbm.at[0, wid, j], sr[j], sem_s[j]).wait()
        pltpu.async_copy(x_hbm.at[sr[j].at[0]], rows[j], semg[j])

    plsc.subcore_barrier()

    lane0 = jnp.zeros((L,), jnp.int32)

    def outer(i, carry):
        for j in range(RING):
            k = i * RING + j
            pltpu.make_async_copy(x_hbm.at[sr[j].at[0]], rows[j], semg[j]).wait()

            def scale_row(r4, inner, j=j, k=k):
                for u in range(4):
                    r = r4 * 4 + u
                    sv = plsc.load_gather(w_loc, [lane0 + (k * CH + r)])
                    for q in range(D // L):
                        rows[j][r, pl.ds(q * L, L)] = (
                            rows[j][r, pl.ds(q * L, L)] * sv)
                return inner

            lax.fori_loop(0, CH // 4, scale_row, 0)
            pltpu.async_copy(rows[j], agg.at[dst2d.at[k, 0]], sems[j], add=True)

            j2 = (j + RING - 2) % RING
            j1 = (j + 2) % RING

            @pl.when(k >= 2)
            def _():
                pltpu.make_async_copy(
                    rows[j2], agg.at[dst2d.at[k - 2, 0]], sems[j2]).wait()

            @pl.when(k + 3 < NCH)
            def _():
                pltpu.async_copy(eidx_hbm.at[0, wid, k + 3], sr[j2], sem_s[j2])

            @pl.when(k + 2 < NCH)
            def _():
                pltpu.make_async_copy(
                    eidx_hbm.at[0, wid, k + 2], sr[j1], sem_s[j1]).wait()
                pltpu.async_copy(x_hbm.at[sr[j1].at[0]], rows[j1], semg[j1])
        return carry

    lax.fori_loop(0, NCH // RING, outer, 0)
    for j in (RING - 2, RING - 1):
        pltpu.make_async_copy(
            rows[j], agg.at[dst2d.at[NCH - RING + j, 0]], sems[j]).wait()
    plsc.subcore_barrier()
    pltpu.sync_copy(agg.at[pl.ds(base, RPT)], out_hbm.at[c, pl.ds(base, RPT)])


BLK = 1000


def _final_body(agg_ref, w_ref, nd_ref, b_ref, out_ref):
    a = agg_ref[0] + agg_ref[1]
    acc = jnp.dot(a, w_ref[...], preferred_element_type=jnp.float32)
    out_ref[...] = acc * nd_ref[...] + b_ref[...]


def _final_tc(aggp, W, ndst, b):
    return pl.pallas_call(
        _final_body,
        grid=(N // BLK,),
        in_specs=[
            pl.BlockSpec((2, BLK, D), lambda i: (0, i, 0)),
            pl.BlockSpec((D, D), lambda i: (0, 0)),
            pl.BlockSpec((BLK, 1), lambda i: (i, 0)),
            pl.BlockSpec((1, D), lambda i: (0, 0)),
        ],
        out_specs=pl.BlockSpec((BLK, D), lambda i: (i, 0)),
        out_shape=jax.ShapeDtypeStruct((N, D), jnp.float32),
    )(aggp, W, ndst, b.reshape(1, D))


def kernel(node_embedding, edge_embedding, edge_index, W, b):
    eidx = edge_index.astype(jnp.int32)
    w3 = edge_embedding.astype(jnp.float32).reshape(NW, EW)

    degp = _degrees_sc(eidx.reshape(2, NW, EW))
    norms = _norms_tc(degp)
    feat = _feat_tc(node_embedding, norms[0].reshape(N, 1))
    aggp = _scatter_sc(feat, eidx.reshape(2, NW, NCH, 1, CH), w3)
    return _final_tc(aggp, W, norms[1].reshape(N, 1), b)


# speedup vs baseline: 1.5079x; 1.0000x over previous
import functools

import jax
import jax.numpy as jnp
from jax import lax
from jax.experimental import pallas as pl
from jax.experimental.pallas import tpu as pltpu
from jax.experimental.pallas import tpu_sc as plsc

N = 10000
E = 320000
D = 128

NC = 2
NS = 16
L = 16
NW = NC * NS
EW = E // NW
CH = 40
NCH = EW // CH
RPT = N // NS

_mesh = plsc.VectorSubcoreMesh(core_axis_name="c", subcore_axis_name="s")
_sc_params = pltpu.CompilerParams(
    needs_layout_passes=False, use_tc_tiling_on_sc=False)


@functools.partial(
    pl.kernel,
    out_type=jax.ShapeDtypeStruct((2, NW, N), jnp.float32),
    mesh=_mesh,
    scratch_types=[
        pltpu.VMEM((EW,), jnp.int32),
        pltpu.VMEM((EW,), jnp.int32),
        pltpu.VMEM((N,), jnp.float32),
        pltpu.VMEM((N,), jnp.float32),
    ],
    compiler_params=_sc_params,
)
def _degrees_sc(eidx_hbm, out_hbm, sidx, didx, dego, degi):
    c = lax.axis_index("c")
    s = lax.axis_index("s")
    wid = c * NS + s
    pltpu.sync_copy(eidx_hbm.at[0, wid], sidx)
    pltpu.sync_copy(eidx_hbm.at[1, wid], didx)

    zero16 = jnp.zeros((L,), jnp.float32)

    def zb(g, carry):
        dego[pl.ds(g * L, L)] = zero16
        degi[pl.ds(g * L, L)] = zero16
        return carry

    lax.fori_loop(0, N // L, zb, 0)

    one16 = jnp.ones((L,), jnp.float32)

    def cb(g, carry):
        plsc.addupdate_scatter(dego, [sidx[pl.ds(g * L, L)]], one16)
        plsc.addupdate_scatter(degi, [didx[pl.ds(g * L, L)]], one16)
        return carry

    lax.fori_loop(0, EW // L, cb, 0)

    pltpu.sync_copy(dego, out_hbm.at[0, wid])
    pltpu.sync_copy(degi, out_hbm.at[1, wid])


def _norms_body(degp_ref, out_ref):
    deg = jnp.sum(degp_ref[...], axis=1)
    out_ref[...] = lax.rsqrt(jnp.clip(deg, 1.0, None))


def _norms_tc(degp):
    return pl.pallas_call(
        _norms_body,
        out_shape=jax.ShapeDtypeStruct((2, N), jnp.float32),
    )(degp)


def _feat_body(x_ref, ns_ref, feat_ref):
    feat_ref[...] = x_ref[...] * ns_ref[...]


def _feat_tc(x, nsrc):
    return pl.pallas_call(
        _feat_body,
        out_shape=jax.ShapeDtypeStruct((N, D), jnp.float32),
    )(x, nsrc)


RING = 5


@functools.partial(
    pl.kernel,
    out_type=jax.ShapeDtypeStruct((NC, N, D), jnp.float32),
    mesh=_mesh,
    scratch_types=(
        [
            pltpu.VMEM_SHARED((N, D), jnp.float32),
            pltpu.VMEM((NCH, 1, CH), jnp.int32),
            pltpu.VMEM((EW,), jnp.float32),
        ]
        + [pltpu.VMEM((CH, D), jnp.float32)] * RING
        + [pltpu.VMEM((1, CH), jnp.int32)] * RING
        + [pltpu.SemaphoreType.DMA] * (3 * RING)
    ),
    compiler_params=_sc_params,
)
def _scatter_sc(x_hbm, eidx_hbm, w_hbm, out_hbm, agg, dst2d, w_loc, *ring):
    rows = ring[:RING]
    sr = ring[RING:2 * RING]
    semg = ring[2 * RING:3 * RING]
    sems = ring[3 * RING:4 * RING]
    sem_s = ring[4 * RING:5 * RING]
    c = lax.axis_index("c")
    s = lax.axis_index("s")
    wid = c * NS + s

    pltpu.sync_copy(eidx_hbm.at[1, wid], dst2d)
    pltpu.sync_copy(w_hbm.at[wid], w_loc)

    zero16 = jnp.zeros((L,), jnp.float32)

    def zrows(i, carry):
        j = i // (D // L)
        q = i % (D // L)
        rows[0][j, pl.ds(q * L, L)] = zero16
        return carry

    lax.fori_loop(0, CH * (D // L), zrows, 0)

    base = s * RPT
    for k in range(RPT // CH):
        pltpu.sync_copy(rows[0], agg.at[pl.ds(base + k * CH, CH)])
    pltpu.sync_copy(rows[0].at[pl.ds(0, RPT % CH)],
                    agg.at[pl.ds(base + (RPT // CH) * CH, RPT % CH)])

    for j in range(3):
        pltpu.async_copy(eidx_hbm.at[0, wid, j], sr[j], sem_s[j])
    for j in range(2):
        pltpu.make_async_copy(eidx_hbm.at[0, wid, j], sr[j], sem_s[j]).wait()
        pltpu.async_copy(x_hbm.at[sr[j].at[0]], rows[j], semg[j])

    plsc.subcore_barrier()

    lane0 = jnp.zeros((L,), jnp.int32)

    def outer(i, carry):
        for j in range(RING):
            k = i * RING + j
            pltpu.make_async_copy(x_hbm.at[sr[j].at[0]], rows[j], semg[j]).wait()

            def scale_row(r4, inner, j=j, k=k):
                for u in range(4):
                    r = r4 * 4 + u
                    sv = plsc.load_gather(w_loc, [lane0 + (k * CH + r)])
                    for q in range(D // L):
                        rows[j][r, pl.ds(q * L, L)] = (
                            rows[j][r, pl.ds(q * L, L)] * sv)
                return inner

            lax.fori_loop(0, CH // 4, scale_row, 0)
            pltpu.async_copy(rows[j], agg.at[dst2d.at[k, 0]], sems[j], add=True)

            j2 = (j + RING - 2) % RING
            j1 = (j + 2) % RING

            @pl.when(k >= 2)
            def _():
                pltpu.make_async_copy(
                    rows[j2], agg.at[dst2d.at[k - 2, 0]], sems[j2]).wait()

            @pl.when(k + 3 < NCH)
            def _():
                pltpu.async_copy(eidx_hbm.at[0, wid, k + 3], sr[j2], sem_s[j2])

            @pl.when(k + 2 < NCH)
            def _():
                pltpu.make_async_copy(
                    eidx_hbm.at[0, wid, k + 2], sr[j1], sem_s[j1]).wait()
                pltpu.async_copy(x_hbm.at[sr[j1].at[0]], rows[j1], semg[j1])
        return carry

    lax.fori_loop(0, NCH // RING, outer, 0)
    for j in (RING - 2, RING - 1):
        pltpu.make_async_copy(
            rows[j], agg.at[dst2d.at[NCH - RING + j, 0]], sems[j]).wait()
    plsc.subcore_barrier()
    pltpu.sync_copy(agg.at[pl.ds(base, RPT)], out_hbm.at[c, pl.ds(base, RPT)])


BLK = 1000


def _final_body(agg_ref, w_ref, nd_ref, b_ref, out_ref):
    a = agg_ref[0] + agg_ref[1]
    acc = jnp.dot(a, w_ref[...], preferred_element_type=jnp.float32)
    out_ref[...] = acc * nd_ref[...] + b_ref[...]


def _final_tc(aggp, W, ndst, b):
    return pl.pallas_call(
        _final_body,
        grid=(N // BLK,),
        in_specs=[
            pl.BlockSpec((2, BLK, D), lambda i: (0, i, 0)),
            pl.BlockSpec((D, D), lambda i: (0, 0)),
            pl.BlockSpec((BLK, 1), lambda i: (i, 0)),
            pl.BlockSpec((1, D), lambda i: (0, 0)),
        ],
        out_specs=pl.BlockSpec((BLK, D), lambda i: (i, 0)),
        out_shape=jax.ShapeDtypeStruct((N, D), jnp.float32),
    )(aggp, W, ndst, b.reshape(1, D))


def kernel(node_embedding, edge_embedding, edge_index, W, b):
    eidx = edge_index.astype(jnp.int32)
    w3 = edge_embedding.astype(jnp.float32).reshape(NW, EW)

    degp = _degrees_sc(eidx.reshape(2, NW, EW))
    norms = _norms_tc(degp)
    feat = _feat_tc(node_embedding, norms[0].reshape(N, 1))
    aggp = _scatter_sc(feat, eidx.reshape(2, NW, NCH, 1, CH), w3)
    return _final_tc(aggp, W, norms[1].reshape(N, 1), b)
